# baseline (speedup 1.0000x reference)
import functools

import jax
import jax.numpy as jnp
from jax import lax
from jax.experimental import pallas as pl
from jax.experimental.pallas import tpu as pltpu
from jax.experimental.pallas import tpu_sc as plsc

B, N, D, M = 8, 2048, 256, 18
R = 256


def _sc_anchor_ego(pm_t, at_p, ea_bc):
    info = plsc.get_sparse_core_info()
    nc = info.num_cores
    mesh = plsc.VectorSubcoreMesh(core_axis_name="c", subcore_axis_name="s")

    @functools.partial(
        pl.kernel,
        mesh=mesh,
        out_type=[
            jax.ShapeDtypeStruct((B, 9, 1, N + M), jnp.float32),
            jax.ShapeDtypeStruct((M, B, D), jnp.float32),
        ],
        scratch_types=[
            pltpu.VMEM((M, D), jnp.float32),
            pltpu.VMEM((9, N + M), jnp.float32),
            pltpu.VMEM((9, 128), jnp.float32),
            pltpu.SemaphoreType.DMA,
        ],
    )
    def k(pm_hbm, at_hbm, ea_hbm, ta_hbm, ef_hbm, pbuf, abuf, tbuf, sq):
        w = lax.axis_index("s") * nc + lax.axis_index("c")
        b = w // 4
        q = w % 4

        @pl.when(q == 0)
        def _anchor():
            pltpu.async_copy(at_hbm.at[:, b, :], abuf.at[:, pl.ds(0, N)], sq).wait()
            pltpu.async_copy(ea_hbm, tbuf, sq).wait()
            for j in range(9):
                sp = tbuf[j, pl.ds(0, 16)]
                abuf[j, pl.ds(N, 16)] = sp
                abuf[j, pl.ds(N + 2, 16)] = sp
            pltpu.async_copy(abuf, ta_hbm.at[b, :, 0, :], sq).wait()

        @pl.when(q == 1)
        def _ef():
            pltpu.async_copy(pm_hbm.at[b, :, 0, :], pbuf, sq).wait()
            pltpu.async_copy(pbuf, ef_hbm.at[:, b, :], sq).wait()

    return k(pm_t, at_p, ea_bc)


def _tc_small(ea):
    def body(ea_ref, mask_ref, eat_ref, bc_ref):
        col = ea_ref[...].reshape(9, 1)
        bc_ref[...] = jnp.broadcast_to(col, (9, 128))
        eat_ref[...] = jnp.broadcast_to(col[:, :, None], (9, B, M))
        mask_ref[...] = jnp.zeros((1, B, N + M), jnp.bool_)

    return pl.pallas_call(
        body,
        out_shape=[
            jax.ShapeDtypeStruct((1, B, N + M), jnp.bool_),
            jax.ShapeDtypeStruct((9, B, M), jnp.float32),
            jax.ShapeDtypeStruct((9, 128), jnp.float32),
        ],
    )(ea)


def _tc_tif(af, pm_t):
    H = N // 2

    def body(af_ref, pm_ref, out_ref, sem, psem):
        b = pl.program_id(0)
        h = pl.program_id(1)

        @pl.when(h == 0)
        def _():
            cpm = pltpu.make_async_copy(
                pm_ref.at[0, :, 0, :], out_ref.at[b, pl.ds(N, M), 0, :], psem)
            cpm.start()
            cpm.wait()

        caf = pltpu.make_async_copy(
            af_ref.at[0], out_ref.at[b, pl.ds(h * H, H), 0, :], sem)
        caf.start()
        caf.wait()

    return pl.pallas_call(
        body,
        grid=(B, 2),
        in_specs=[
            pl.BlockSpec((1, H, D), lambda b, h: (b, h, 0)),
            pl.BlockSpec((1, M, 1, D), lambda b, h: (b, 0, 0, 0)),
        ],
        out_specs=pl.BlockSpec(memory_space=pl.ANY),
        out_shape=jax.ShapeDtypeStruct((B, N + M, 1, D), jnp.float32),
        scratch_shapes=[pltpu.SemaphoreType.DMA, pltpu.SemaphoreType.DMA],
    )(af, pm_t)


def kernel(agent_target, agent_feature, agent_mask, plan_mode_query, ego_anchor, batch_size):
    pm_t = plan_mode_query.transpose(0, 2, 1, 3)
    at_p = agent_target.transpose(2, 0, 1)

    mask_p, eat_p, ea_bc = _tc_small(ego_anchor)
    ta_p, ef_p = _sc_anchor_ego(pm_t, at_p, ea_bc)
    tif = _tc_tif(agent_feature, pm_t)

    ego_feature = ef_p.transpose(1, 0, 2)
    ego_anchor_t = eat_p.transpose(1, 2, 0)
    temp_anchor = ta_p.transpose(0, 3, 2, 1)
    temp_mask = mask_p.transpose(1, 2, 0)
    return (ego_feature, ego_anchor_t, tif, temp_anchor, temp_mask)


# speedup vs baseline: 1.3393x; 1.3393x over previous
import functools

import jax
import jax.numpy as jnp
from jax import lax
from jax.experimental import pallas as pl
from jax.experimental.pallas import tpu as pltpu
from jax.experimental.pallas import tpu_sc as plsc

B, N, D, M = 8, 2048, 256, 18
R = 256


def _sc_anchor_ego(pm_t, at_p, ea_bc):
    info = plsc.get_sparse_core_info()
    nc = info.num_cores
    mesh = plsc.VectorSubcoreMesh(core_axis_name="c", subcore_axis_name="s")

    @functools.partial(
        pl.kernel,
        mesh=mesh,
        out_type=[
            jax.ShapeDtypeStruct((B, 9, 1, N + M), jnp.float32),
            jax.ShapeDtypeStruct((M, B, D), jnp.float32),
        ],
        scratch_types=[
            pltpu.VMEM((M, D), jnp.float32),
            pltpu.VMEM((9, N + M), jnp.float32),
            pltpu.VMEM((9, 128), jnp.float32),
            pltpu.SemaphoreType.DMA,
        ],
    )
    def k(pm_hbm, at_hbm, ea_hbm, ta_hbm, ef_hbm, pbuf, abuf, tbuf, sq):
        w = lax.axis_index("s") * nc + lax.axis_index("c")
        b = w // 4
        q = w % 4

        @pl.when(q == 0)
        def _anchor():
            pltpu.async_copy(at_hbm.at[:, b, :], abuf.at[:, pl.ds(0, N)], sq).wait()
            pltpu.async_copy(ea_hbm, tbuf, sq).wait()
            for j in range(9):
                sp = tbuf[j, pl.ds(0, 16)]
                abuf[j, pl.ds(N, 16)] = sp
                abuf[j, pl.ds(N + 2, 16)] = sp
            pltpu.async_copy(abuf, ta_hbm.at[b, :, 0, :], sq).wait()

        @pl.when(q == 1)
        def _ef():
            pltpu.async_copy(pm_hbm.at[b, :, 0, :], pbuf, sq).wait()
            pltpu.async_copy(pbuf, ef_hbm.at[:, b, :], sq).wait()

    return k(pm_t, at_p, ea_bc)


def _tc_small(ea):
    def body(ea_ref, mask_ref, eat_ref, bc_ref):
        col = ea_ref[...].reshape(9, 1)
        bc_ref[...] = jnp.broadcast_to(col, (9, 128))
        eat_ref[...] = jnp.broadcast_to(col[:, :, None], (9, B, M))
        mask_ref[...] = jnp.zeros((1, B, N + M), jnp.bool_)

    return pl.pallas_call(
        body,
        out_shape=[
            jax.ShapeDtypeStruct((1, B, N + M), jnp.bool_),
            jax.ShapeDtypeStruct((9, B, M), jnp.float32),
            jax.ShapeDtypeStruct((9, 128), jnp.float32),
        ],
    )(ea)


def _tc_tif(af, pm_t):
    H = N // 2

    def body(af_ref, pm_ref, out_ref, sem0, sem1, psem):
        b = pl.program_id(0)
        cpm = pltpu.make_async_copy(
            pm_ref.at[0, :, 0, :], out_ref.at[b, pl.ds(N, M), 0, :], psem)
        cpm.start()
        c0 = pltpu.make_async_copy(
            af_ref.at[0, pl.ds(0, H), :], out_ref.at[b, pl.ds(0, H), 0, :], sem0)
        c1 = pltpu.make_async_copy(
            af_ref.at[0, pl.ds(H, H), :], out_ref.at[b, pl.ds(H, H), 0, :], sem1)
        c0.start()
        c1.start()
        c0.wait()
        c1.wait()
        cpm.wait()

    return pl.pallas_call(
        body,
        grid=(B,),
        in_specs=[
            pl.BlockSpec((1, N, D), lambda b: (b, 0, 0)),
            pl.BlockSpec((1, M, 1, D), lambda b: (b, 0, 0, 0)),
        ],
        out_specs=pl.BlockSpec(memory_space=pl.ANY),
        out_shape=jax.ShapeDtypeStruct((B, N + M, 1, D), jnp.float32),
        scratch_shapes=[pltpu.SemaphoreType.DMA, pltpu.SemaphoreType.DMA,
                        pltpu.SemaphoreType.DMA],
    )(af, pm_t)


def kernel(agent_target, agent_feature, agent_mask, plan_mode_query, ego_anchor, batch_size):
    pm_t = plan_mode_query.transpose(0, 2, 1, 3)
    at_p = agent_target.transpose(2, 0, 1)

    mask_p, eat_p, ea_bc = _tc_small(ego_anchor)
    ta_p, ef_p = _sc_anchor_ego(pm_t, at_p, ea_bc)
    tif = _tc_tif(agent_feature, pm_t)

    ego_feature = ef_p.transpose(1, 0, 2)
    ego_anchor_t = eat_p.transpose(1, 2, 0)
    temp_anchor = ta_p.transpose(0, 3, 2, 1)
    temp_mask = mask_p.transpose(1, 2, 0)
    return (ego_feature, ego_anchor_t, tif, temp_anchor, temp_mask)
